# baseline (speedup 1.0000x reference)
import functools
import math

import jax
import jax.numpy as jnp
from jax import lax
from jax.experimental import pallas as pl
from jax.experimental.pallas import tpu as pltpu
from jax.experimental.pallas import tpu_sc as plsc

B, L, V, D, P = 4096, 200, 100000, 128, 512
N = B * L
NW = 32
TOK_PER_W = N // NW
CHUNK = 128
N_CHUNKS = TOK_PER_W // CHUNK
K = D // 16
EPS = 1e-5 / D


def _rsqrt(v):
    bits = plsc.bitcast(v, jnp.int32)
    y = plsc.bitcast(jnp.int32(0x5F3759DF) - (bits >> 1), jnp.float32)
    for _ in range(2):
        y = y * (1.5 - 0.5 * v * y * y)
    return y


_mesh = plsc.VectorSubcoreMesh(core_axis_name="c", subcore_axis_name="s")


@functools.partial(
    pl.kernel,
    mesh=_mesh,
    out_type=jax.ShapeDtypeStruct((N, D), jnp.float32),
    compiler_params=pltpu.CompilerParams(needs_layout_passes=False),
    scratch_types=[
        pltpu.VMEM((P * D,), jnp.float32),
        pltpu.VMEM((2, CHUNK), jnp.int32),
        pltpu.VMEM((2, CHUNK), jnp.int32),
        pltpu.VMEM((2, CHUNK, D), jnp.float32),
        pltpu.VMEM((D,), jnp.float32),
        pltpu.VMEM((D,), jnp.float32),
        pltpu.SemaphoreType.DMA,
        pltpu.SemaphoreType.DMA,
        pltpu.SemaphoreType.DMA,
        pltpu.SemaphoreType.DMA,
        pltpu.SemaphoreType.DMA,
        pltpu.SemaphoreType.DMA,
    ],
)
def _embed_ln(ids_hbm, ord_hbm, wtab_hbm, pe_hbm, gamma_hbm, beta_hbm,
              out_hbm, pe_v, ids_v, ord_v, x_v, g_v, b_v,
              si0, si1, sg0, sg1, so0, so1):
    sem_i = (si0, si1)
    sem_g = (sg0, sg1)
    sem_o = (so0, so1)
    wid = lax.axis_index("s") * 2 + lax.axis_index("c")
    base0 = wid * TOK_PER_W
    pltpu.sync_copy(pe_hbm, pe_v)
    pltpu.sync_copy(gamma_hbm, g_v)
    pltpu.sync_copy(beta_hbm, b_v)
    gk = [g_v[pl.ds(k * 16, 16)] for k in range(K)]
    bk = [b_v[pl.ds(k * 16, 16)] for k in range(K)]

    def issue_ids(ci, p):
        base = base0 + ci * CHUNK
        pltpu.async_copy(ids_hbm.at[pl.ds(base, CHUNK)], ids_v.at[p], sem_i[p])
        pltpu.async_copy(ord_hbm.at[pl.ds(base, CHUNK)], ord_v.at[p], sem_i[p])

    def wait_ids(p):
        pltpu.make_async_copy(ids_hbm.at[pl.ds(0, CHUNK)], ids_v.at[p],
                              sem_i[p]).wait()
        pltpu.make_async_copy(ord_hbm.at[pl.ds(0, CHUNK)], ord_v.at[p],
                              sem_i[p]).wait()

    def issue_gather(p):
        pltpu.async_copy(wtab_hbm.at[ids_v.at[p]], x_v.at[p], sem_g[p])

    def wait_gather(p):
        pltpu.make_async_copy(wtab_hbm.at[ids_v.at[p]], x_v.at[p],
                              sem_g[p]).wait()

    def wait_out(p):
        pltpu.make_async_copy(x_v.at[p], out_hbm.at[pl.ds(0, CHUNK)],
                              sem_o[p]).wait()

    issue_ids(0, 0)
    issue_ids(1, 1)
    wait_ids(0)
    issue_gather(0)

    def outer(ii, _):
        for p in range(2):
            ci = 2 * ii + p
            wait_gather(p)

            @pl.when(ci + 1 < N_CHUNKS)
            def _():
                @pl.when(ci >= 1)
                def _():
                    wait_out(1 - p)

                wait_ids(1 - p)
                issue_gather(1 - p)

            def grp(gi, c):
                ov = ord_v[p, pl.ds(gi * 16, 16)]
                for j in range(16):
                    t = gi * 16 + j
                    po = ov[j] * D
                    x = [
                        x_v[p, t, pl.ds(k * 16, 16)]
                        + pe_v[pl.ds(po + k * 16, 16)]
                        for k in range(K)
                    ]
                    s = x[0]
                    q = x[0] * x[0]
                    for k in range(1, K):
                        s = s + x[k]
                        q = q + x[k] * x[k]
                    tot = jnp.sum(s)
                    totq = jnp.sum(q)
                    mean = tot * (1.0 / D)
                    var = totq * (1.0 / D) - mean * mean
                    mean16 = jnp.full((16,), mean, jnp.float32)
                    rstd16 = _rsqrt(jnp.full((16,), var + EPS, jnp.float32))
                    for k in range(K):
                        y = (x[k] - mean16) * rstd16 * gk[k] + bk[k]
                        x_v[p, t, pl.ds(k * 16, 16)] = y
                return c

            lax.fori_loop(0, CHUNK // 16, grp, 0)

            @pl.when(ci + 2 < N_CHUNKS)
            def _():
                issue_ids(ci + 2, p)

            base = base0 + ci * CHUNK
            pltpu.async_copy(x_v.at[p], out_hbm.at[pl.ds(base, CHUNK)],
                             sem_o[p])
        return 0

    lax.fori_loop(0, N_CHUNKS // 2, outer, 0)
    wait_out(0)
    wait_out(1)


def kernel(masked_token_ids, visit_concept_orders, word_embeddings, pe,
           gamma, beta):
    ids = masked_token_ids.reshape(-1).astype(jnp.int32)
    orders = visit_concept_orders.reshape(-1).astype(jnp.int32)
    pe_s = (pe * (1.0 / math.sqrt(D))).astype(jnp.float32).reshape(-1)
    out = _embed_ln(ids, orders, word_embeddings, pe_s,
                    gamma.astype(jnp.float32), beta.astype(jnp.float32))
    return out.reshape(masked_token_ids.shape[0], masked_token_ids.shape[1], D)


# speedup vs baseline: 1.2517x; 1.2517x over previous
import functools
import math

import jax
import jax.numpy as jnp
from jax import lax
from jax.experimental import pallas as pl
from jax.experimental.pallas import tpu as pltpu
from jax.experimental.pallas import tpu_sc as plsc

B, L, V, D, P = 4096, 200, 100000, 128, 512
N = B * L
NW = 32
TOK_PER_W = N // NW
CHUNK = 128
N_CHUNKS = TOK_PER_W // CHUNK
K = D // 16
EPS = 1e-5 / D
TPI = 8


def _rsqrt(v):
    bits = plsc.bitcast(v, jnp.int32)
    y = plsc.bitcast(jnp.int32(0x5F3759DF) - (bits >> 1), jnp.float32)
    return y * (1.5 - 0.5 * v * y * y)


_mesh = plsc.VectorSubcoreMesh(core_axis_name="c", subcore_axis_name="s")


@functools.partial(
    pl.kernel,
    mesh=_mesh,
    out_type=jax.ShapeDtypeStruct((N, D), jnp.float32),
    compiler_params=pltpu.CompilerParams(needs_layout_passes=False),
    scratch_types=[
        pltpu.VMEM((P * D,), jnp.float32),
        pltpu.VMEM((2, CHUNK), jnp.int32),
        pltpu.VMEM((2, CHUNK + TPI), jnp.int32),
        pltpu.VMEM((2, CHUNK, D), jnp.float32),
        pltpu.VMEM((D,), jnp.float32),
        pltpu.VMEM((D,), jnp.float32),
        pltpu.SemaphoreType.DMA,
        pltpu.SemaphoreType.DMA,
        pltpu.SemaphoreType.DMA,
        pltpu.SemaphoreType.DMA,
        pltpu.SemaphoreType.DMA,
        pltpu.SemaphoreType.DMA,
    ],
)
def _embed_ln(ids_hbm, ord_hbm, wtab_hbm, pe_hbm, gamma_hbm, beta_hbm,
              out_hbm, pe_v, ids_v, ord_v, x_v, g_v, b_v,
              si0, si1, sg0, sg1, so0, so1):
    sem_i = (si0, si1)
    sem_g = (sg0, sg1)
    sem_o = (so0, so1)
    wid = lax.axis_index("s") * 2 + lax.axis_index("c")
    base0 = wid * TOK_PER_W
    pltpu.sync_copy(pe_hbm, pe_v)
    pltpu.sync_copy(gamma_hbm, g_v)
    pltpu.sync_copy(beta_hbm, b_v)
    gk = [g_v[pl.ds(k * 16, 16)] for k in range(K)]
    bk = [b_v[pl.ds(k * 16, 16)] for k in range(K)]
    lane = lax.iota(jnp.int32, 16)
    perms = [lane ^ st for st in (1, 2, 4, 8)]

    def _lanesum(v):
        for pm in perms:
            v = v + jnp.take_along_axis(v, pm, axis=0,
                                        mode="promise_in_bounds")
        return v

    def issue_ids(ci, p):
        base = base0 + ci * CHUNK
        pltpu.async_copy(ids_hbm.at[pl.ds(base, CHUNK)], ids_v.at[p], sem_i[p])
        pltpu.async_copy(ord_hbm.at[pl.ds(base, CHUNK)],
                         ord_v.at[p, pl.ds(0, CHUNK)], sem_i[p])

    def wait_ids(p):
        pltpu.make_async_copy(ids_hbm.at[pl.ds(0, CHUNK)], ids_v.at[p],
                              sem_i[p]).wait()
        pltpu.make_async_copy(ord_hbm.at[pl.ds(0, CHUNK)],
                              ord_v.at[p, pl.ds(0, CHUNK)], sem_i[p]).wait()

    def issue_gather(p):
        pltpu.async_copy(wtab_hbm.at[ids_v.at[p]], x_v.at[p], sem_g[p])

    def wait_gather(p):
        pltpu.make_async_copy(wtab_hbm.at[ids_v.at[p]], x_v.at[p],
                              sem_g[p]).wait()

    def wait_out(p):
        pltpu.make_async_copy(x_v.at[p], out_hbm.at[pl.ds(0, CHUNK)],
                              sem_o[p]).wait()

    issue_ids(0, 0)
    issue_ids(1, 1)
    wait_ids(0)
    issue_gather(0)

    def outer(ii, _):
        for p in range(2):
            ci = 2 * ii + p
            wait_gather(p)

            @pl.when(ci + 1 < N_CHUNKS)
            def _():
                @pl.when(ci >= 1)
                def _():
                    wait_out(1 - p)

                wait_ids(1 - p)
                issue_gather(1 - p)

            def grp(gi, c):
                ov = ord_v[p, pl.ds(gi * TPI, 16)]
                for j in range(TPI):
                    t = gi * TPI + j
                    po = ov[j]
                    x = [
                        x_v[p, t, pl.ds(k * 16, 16)]
                        + pe_v[pl.ds(po + k * 16, 16)]
                        for k in range(K)
                    ]
                    s = x[0]
                    q = x[0] * x[0]
                    for k in range(1, K):
                        s = s + x[k]
                        q = q + x[k] * x[k]
                    mean16 = _lanesum(s) * (1.0 / D)
                    var16 = _lanesum(q) * (1.0 / D) - mean16 * mean16
                    rstd16 = _rsqrt(var16 + EPS)
                    for k in range(K):
                        y = (x[k] - mean16) * rstd16 * gk[k] + bk[k]
                        x_v[p, t, pl.ds(k * 16, 16)] = y
                return c

            lax.fori_loop(0, CHUNK // TPI, grp, 0)

            @pl.when(ci + 2 < N_CHUNKS)
            def _():
                issue_ids(ci + 2, p)

            base = base0 + ci * CHUNK
            pltpu.async_copy(x_v.at[p], out_hbm.at[pl.ds(base, CHUNK)],
                             sem_o[p])
        return 0

    lax.fori_loop(0, N_CHUNKS // 2, outer, 0)
    wait_out(0)
    wait_out(1)


def kernel(masked_token_ids, visit_concept_orders, word_embeddings, pe,
           gamma, beta):
    ids = masked_token_ids.reshape(-1).astype(jnp.int32)
    orders = visit_concept_orders.reshape(-1).astype(jnp.int32) * D
    pe_s = (pe * (1.0 / math.sqrt(D))).astype(jnp.float32).reshape(-1)
    out = _embed_ln(ids, orders, word_embeddings, pe_s,
                    gamma.astype(jnp.float32), beta.astype(jnp.float32))
    return out.reshape(masked_token_ids.shape[0], masked_token_ids.shape[1], D)


# speedup vs baseline: 1.6998x; 1.3580x over previous
import functools
import math

import jax
import jax.numpy as jnp
from jax import lax
from jax.experimental import pallas as pl
from jax.experimental.pallas import tpu as pltpu
from jax.experimental.pallas import tpu_sc as plsc

B, L, V, D, P = 4096, 200, 100000, 128, 512
N = B * L
NW = 32
TOK_PER_W = N // NW
CHUNK = 80
N_CHUNKS = TOK_PER_W // CHUNK
K = D // 16
EPS = 1e-5 / D
TPI = 4


def _rsqrt(v):
    bits = plsc.bitcast(v, jnp.int32)
    y = plsc.bitcast(jnp.int32(0x5F3759DF) - (bits >> 1), jnp.float32)
    return y * (1.5 - 0.5 * v * y * y)


_mesh = plsc.VectorSubcoreMesh(core_axis_name="c", subcore_axis_name="s")


@functools.partial(
    pl.kernel,
    mesh=_mesh,
    out_type=jax.ShapeDtypeStruct((N, D), jnp.float32),
    compiler_params=pltpu.CompilerParams(needs_layout_passes=False),
    scratch_types=[
        pltpu.VMEM((P * D,), jnp.float32),
        pltpu.VMEM((2, CHUNK), jnp.int32),
        pltpu.VMEM((2, CHUNK + 16), jnp.int32),
        pltpu.VMEM((2, CHUNK, D), jnp.float32),
        pltpu.VMEM((2, CHUNK, D), jnp.float32),
        pltpu.VMEM((D,), jnp.float32),
        pltpu.VMEM((D,), jnp.float32),
        pltpu.SemaphoreType.DMA,
        pltpu.SemaphoreType.DMA,
        pltpu.SemaphoreType.DMA,
        pltpu.SemaphoreType.DMA,
        pltpu.SemaphoreType.DMA,
        pltpu.SemaphoreType.DMA,
    ],
)
def _embed_ln(ids_hbm, ord_hbm, wtab_hbm, pe_hbm, gamma_hbm, beta_hbm,
              out_hbm, pe_v, ids_v, ord_v, x_v, o_v, g_v, b_v,
              si0, si1, sg0, sg1, so0, so1):
    sem_i = (si0, si1)
    sem_g = (sg0, sg1)
    sem_o = (so0, so1)
    wid = lax.axis_index("s") * 2 + lax.axis_index("c")
    base0 = wid * TOK_PER_W
    pltpu.sync_copy(pe_hbm, pe_v)
    pltpu.sync_copy(gamma_hbm, g_v)
    pltpu.sync_copy(beta_hbm, b_v)
    gk = [g_v[pl.ds(k * 16, 16)] for k in range(K)]
    bk = [b_v[pl.ds(k * 16, 16)] for k in range(K)]
    lane = lax.iota(jnp.int32, 16)
    perms = [lane ^ st for st in (1, 2, 4, 8)]

    def _lanesum(v):
        for pm in perms:
            v = v + jnp.take_along_axis(v, pm, axis=0,
                                        mode="promise_in_bounds")
        return v

    def issue_ids(ci, p):
        base = base0 + ci * CHUNK
        pltpu.async_copy(ids_hbm.at[pl.ds(base, CHUNK)], ids_v.at[p], sem_i[p])
        pltpu.async_copy(ord_hbm.at[pl.ds(base, CHUNK)],
                         ord_v.at[p, pl.ds(0, CHUNK)], sem_i[p])

    def wait_ids(p):
        pltpu.make_async_copy(ids_hbm.at[pl.ds(0, CHUNK)], ids_v.at[p],
                              sem_i[p]).wait()
        pltpu.make_async_copy(ord_hbm.at[pl.ds(0, CHUNK)],
                              ord_v.at[p, pl.ds(0, CHUNK)], sem_i[p]).wait()

    def issue_gather(p):
        pltpu.async_copy(wtab_hbm.at[ids_v.at[p]], x_v.at[p], sem_g[p])

    def wait_gather(p):
        pltpu.make_async_copy(wtab_hbm.at[ids_v.at[p]], x_v.at[p],
                              sem_g[p]).wait()

    def wait_out(p):
        pltpu.make_async_copy(o_v.at[p], out_hbm.at[pl.ds(0, CHUNK)],
                              sem_o[p]).wait()

    issue_ids(0, 0)
    issue_ids(1, 1)
    wait_ids(0)
    issue_gather(0)

    def outer(ii, _):
        for p in range(2):
            ci = 2 * ii + p
            wait_gather(p)

            @pl.when(ci + 1 < N_CHUNKS)
            def _():
                wait_ids(1 - p)
                issue_gather(1 - p)

            @pl.when(ci >= 2)
            def _():
                wait_out(p)

            def grp(gi, c):
                ov = ord_v[p, pl.ds(gi * TPI, 16)]
                po = [ov[j] for j in range(TPI)]
                xs = []
                for j in range(TPI):
                    t = gi * TPI + j
                    xs.append([
                        x_v[p, t, pl.ds(k * 16, 16)]
                        + pe_v[pl.ds(po[j] + k * 16, 16)]
                        for k in range(K)
                    ])
                ss, qs = [], []
                for j in range(TPI):
                    s = xs[j]
                    q = [xk * xk for xk in xs[j]]
                    while len(s) > 1:
                        s = [a + b for a, b in zip(s[::2], s[1::2])]
                        q = [a + b for a, b in zip(q[::2], q[1::2])]
                    ss.append(s[0])
                    qs.append(q[0])
                ss = [_lanesum(v) for v in ss]
                qs = [_lanesum(v) for v in qs]
                means = [v * (1.0 / D) for v in ss]
                rstds = [
                    _rsqrt(qs[j] * (1.0 / D) - means[j] * means[j] + EPS)
                    for j in range(TPI)
                ]
                for j in range(TPI):
                    t = gi * TPI + j
                    for k in range(K):
                        y = (xs[j][k] - means[j]) * rstds[j] * gk[k] + bk[k]
                        o_v[p, t, pl.ds(k * 16, 16)] = y
                return c

            lax.fori_loop(0, CHUNK // TPI, grp, 0)

            @pl.when(ci + 2 < N_CHUNKS)
            def _():
                issue_ids(ci + 2, p)

            base = base0 + ci * CHUNK
            pltpu.async_copy(o_v.at[p], out_hbm.at[pl.ds(base, CHUNK)],
                             sem_o[p])
        return 0

    lax.fori_loop(0, N_CHUNKS // 2, outer, 0)
    wait_out(0)
    wait_out(1)


def kernel(masked_token_ids, visit_concept_orders, word_embeddings, pe,
           gamma, beta):
    ids = masked_token_ids.reshape(-1).astype(jnp.int32)
    orders = visit_concept_orders.reshape(-1).astype(jnp.int32) * D
    pe_s = (pe * (1.0 / math.sqrt(D))).astype(jnp.float32).reshape(-1)
    out = _embed_ln(ids, orders, word_embeddings, pe_s,
                    gamma.astype(jnp.float32), beta.astype(jnp.float32))
    return out.reshape(masked_token_ids.shape[0], masked_token_ids.shape[1], D)


# speedup vs baseline: 1.7307x; 1.0182x over previous
import functools
import math

import jax
import jax.numpy as jnp
from jax import lax
from jax.experimental import pallas as pl
from jax.experimental.pallas import tpu as pltpu
from jax.experimental.pallas import tpu_sc as plsc

B, L, V, D, P = 4096, 200, 100000, 128, 512
N = B * L
NW = 32
TOK_PER_W = N // NW
CHUNK = 80
N_CHUNKS = TOK_PER_W // CHUNK
K = D // 16
EPS = 1e-5 / D
TPI = 5


def _rsqrt(v):
    bits = plsc.bitcast(v, jnp.int32)
    y = plsc.bitcast(jnp.int32(0x5F3759DF) - (bits >> 1), jnp.float32)
    return y * (1.5 - 0.5 * v * y * y)


_mesh = plsc.VectorSubcoreMesh(core_axis_name="c", subcore_axis_name="s")


@functools.partial(
    pl.kernel,
    mesh=_mesh,
    out_type=jax.ShapeDtypeStruct((N, D), jnp.float32),
    compiler_params=pltpu.CompilerParams(needs_layout_passes=False),
    scratch_types=[
        pltpu.VMEM((P * D,), jnp.float32),
        pltpu.VMEM((2, CHUNK), jnp.int32),
        pltpu.VMEM((2, CHUNK + 16), jnp.int32),
        pltpu.VMEM((2, CHUNK, D), jnp.float32),
        pltpu.VMEM((2, CHUNK, D), jnp.float32),
        pltpu.VMEM((D,), jnp.float32),
        pltpu.VMEM((D,), jnp.float32),
        pltpu.SemaphoreType.DMA,
        pltpu.SemaphoreType.DMA,
        pltpu.SemaphoreType.DMA,
        pltpu.SemaphoreType.DMA,
        pltpu.SemaphoreType.DMA,
        pltpu.SemaphoreType.DMA,
    ],
)
def _embed_ln(ids_hbm, ord_hbm, wtab_hbm, pe_hbm, gamma_hbm, beta_hbm,
              out_hbm, pe_v, ids_v, ord_v, x_v, o_v, g_v, b_v,
              si0, si1, sg0, sg1, so0, so1):
    sem_i = (si0, si1)
    sem_g = (sg0, sg1)
    sem_o = (so0, so1)
    wid = lax.axis_index("s") * 2 + lax.axis_index("c")
    base0 = wid * TOK_PER_W
    pltpu.sync_copy(pe_hbm, pe_v)
    pltpu.sync_copy(gamma_hbm, g_v)
    pltpu.sync_copy(beta_hbm, b_v)
    lane = lax.iota(jnp.int32, 16)
    perms = [lane ^ st for st in (1, 2, 4, 8)]

    def _lanesum(v):
        for pm in perms:
            v = v + jnp.take_along_axis(v, pm, axis=0,
                                        mode="promise_in_bounds")
        return v

    def issue_ids(ci, p):
        base = base0 + ci * CHUNK
        pltpu.async_copy(ids_hbm.at[pl.ds(base, CHUNK)], ids_v.at[p], sem_i[p])
        pltpu.async_copy(ord_hbm.at[pl.ds(base, CHUNK)],
                         ord_v.at[p, pl.ds(0, CHUNK)], sem_i[p])

    def wait_ids(p):
        pltpu.make_async_copy(ids_hbm.at[pl.ds(0, CHUNK)], ids_v.at[p],
                              sem_i[p]).wait()
        pltpu.make_async_copy(ord_hbm.at[pl.ds(0, CHUNK)],
                              ord_v.at[p, pl.ds(0, CHUNK)], sem_i[p]).wait()

    def issue_gather(p):
        pltpu.async_copy(wtab_hbm.at[ids_v.at[p]], x_v.at[p], sem_g[p])

    def wait_gather(p):
        pltpu.make_async_copy(wtab_hbm.at[ids_v.at[p]], x_v.at[p],
                              sem_g[p]).wait()

    def wait_out(p):
        pltpu.make_async_copy(o_v.at[p], out_hbm.at[pl.ds(0, CHUNK)],
                              sem_o[p]).wait()

    issue_ids(0, 0)
    issue_ids(1, 1)
    wait_ids(0)
    issue_gather(0)

    def outer(ii, _):
        for p in range(2):
            ci = 2 * ii + p
            wait_gather(p)

            @pl.when(ci + 1 < N_CHUNKS)
            def _():
                wait_ids(1 - p)
                issue_gather(1 - p)

            @pl.when(ci >= 2)
            def _():
                wait_out(p)

            def grp(gi, c):
                ov = ord_v[p, pl.ds(gi * TPI, 16)]
                po = [ov[j] for j in range(TPI)]
                gk = [g_v[pl.ds(k * 16, 16)] for k in range(K)]
                bk = [b_v[pl.ds(k * 16, 16)] for k in range(K)]
                xs = []
                for j in range(TPI):
                    t = gi * TPI + j
                    xs.append([
                        x_v[p, t, pl.ds(k * 16, 16)]
                        + pe_v[pl.ds(po[j] + k * 16, 16)]
                        for k in range(K)
                    ])
                ss, qs = [], []
                for j in range(TPI):
                    s = xs[j]
                    q = [xk * xk for xk in xs[j]]
                    while len(s) > 1:
                        s = [a + b for a, b in zip(s[::2], s[1::2])]
                        q = [a + b for a, b in zip(q[::2], q[1::2])]
                    ss.append(s[0])
                    qs.append(q[0])
                ss = [_lanesum(v) for v in ss]
                qs = [_lanesum(v) for v in qs]
                means = [v * (1.0 / D) for v in ss]
                rstds = [
                    _rsqrt(qs[j] * (1.0 / D) - means[j] * means[j] + EPS)
                    for j in range(TPI)
                ]
                for j in range(TPI):
                    t = gi * TPI + j
                    for k in range(K):
                        y = (xs[j][k] - means[j]) * rstds[j] * gk[k] + bk[k]
                        o_v[p, t, pl.ds(k * 16, 16)] = y
                return c

            lax.fori_loop(0, CHUNK // TPI, grp, 0)

            @pl.when(ci + 2 < N_CHUNKS)
            def _():
                issue_ids(ci + 2, p)

            base = base0 + ci * CHUNK
            pltpu.async_copy(o_v.at[p], out_hbm.at[pl.ds(base, CHUNK)],
                             sem_o[p])
        return 0

    lax.fori_loop(0, N_CHUNKS // 2, outer, 0)
    wait_out(0)
    wait_out(1)


def kernel(masked_token_ids, visit_concept_orders, word_embeddings, pe,
           gamma, beta):
    ids = masked_token_ids.reshape(-1).astype(jnp.int32)
    orders = visit_concept_orders.reshape(-1).astype(jnp.int32) * D
    pe_s = (pe * (1.0 / math.sqrt(D))).astype(jnp.float32).reshape(-1)
    out = _embed_ln(ids, orders, word_embeddings, pe_s,
                    gamma.astype(jnp.float32), beta.astype(jnp.float32))
    return out.reshape(masked_token_ids.shape[0], masked_token_ids.shape[1], D)


# speedup vs baseline: 1.9395x; 1.1207x over previous
import functools
import math

import jax
import jax.numpy as jnp
from jax import lax
from jax.experimental import pallas as pl
from jax.experimental.pallas import tpu as pltpu
from jax.experimental.pallas import tpu_sc as plsc

B, L, V, D, P = 4096, 200, 100000, 128, 512
N = B * L
NW = 32
TOK_PER_W = N // NW
CHUNK = 128
N_CHUNKS = TOK_PER_W // CHUNK
K = D // 16
EPS = 1e-5 / D
U = 2


def _rsqrt(v):
    bits = plsc.bitcast(v, jnp.int32)
    y = plsc.bitcast(jnp.int32(0x5F3759DF) - (bits >> 1), jnp.float32)
    for _ in range(3):
        y = y * (1.5 - 0.5 * v * y * y)
    return y


_mesh = plsc.VectorSubcoreMesh(core_axis_name="c", subcore_axis_name="s")


@functools.partial(
    pl.kernel,
    mesh=_mesh,
    out_type=jax.ShapeDtypeStruct((N, D), jnp.float32),
    compiler_params=pltpu.CompilerParams(needs_layout_passes=False),
    scratch_types=[
        pltpu.VMEM((2, CHUNK), jnp.int32),
        pltpu.VMEM((2, CHUNK), jnp.int32),
        pltpu.VMEM((2, CHUNK, D), jnp.float32),
        pltpu.VMEM((2, CHUNK, D), jnp.float32),
        pltpu.VMEM((2, CHUNK, D), jnp.float32),
        pltpu.VMEM((D,), jnp.float32),
        pltpu.VMEM((D,), jnp.float32),
        pltpu.SemaphoreType.DMA,
        pltpu.SemaphoreType.DMA,
        pltpu.SemaphoreType.DMA,
        pltpu.SemaphoreType.DMA,
        pltpu.SemaphoreType.DMA,
        pltpu.SemaphoreType.DMA,
    ],
)
def _embed_ln(ids_hbm, ord_hbm, wtab_hbm, pe_hbm, gamma_hbm, beta_hbm,
              out_hbm, ids_v, ord_v, x_v, p_v, o_v, g_v, b_v,
              si0, si1, sg0, sg1, so0, so1):
    sem_i = (si0, si1)
    sem_g = (sg0, sg1)
    sem_o = (so0, so1)
    wid = lax.axis_index("s") * 2 + lax.axis_index("c")
    base0 = wid * TOK_PER_W
    pltpu.sync_copy(gamma_hbm, g_v)
    pltpu.sync_copy(beta_hbm, b_v)
    gk = [g_v[pl.ds(k * 16, 16)] for k in range(K)]
    bk = [b_v[pl.ds(k * 16, 16)] for k in range(K)]

    def issue_ids(ci, p):
        base = base0 + ci * CHUNK
        pltpu.async_copy(ids_hbm.at[pl.ds(base, CHUNK)], ids_v.at[p], sem_i[p])
        pltpu.async_copy(ord_hbm.at[pl.ds(base, CHUNK)], ord_v.at[p], sem_i[p])

    def wait_ids(p):
        pltpu.make_async_copy(ids_hbm.at[pl.ds(0, CHUNK)], ids_v.at[p],
                              sem_i[p]).wait()
        pltpu.make_async_copy(ord_hbm.at[pl.ds(0, CHUNK)], ord_v.at[p],
                              sem_i[p]).wait()

    def issue_gathers(p):
        pltpu.async_copy(wtab_hbm.at[ids_v.at[p]], x_v.at[p], sem_g[p])
        pltpu.async_copy(pe_hbm.at[ord_v.at[p]], p_v.at[p], sem_g[p])

    def wait_gathers(p):
        pltpu.make_async_copy(wtab_hbm.at[ids_v.at[p]], x_v.at[p],
                              sem_g[p]).wait()
        pltpu.make_async_copy(pe_hbm.at[ord_v.at[p]], p_v.at[p],
                              sem_g[p]).wait()

    def wait_out(p):
        pltpu.make_async_copy(o_v.at[p], out_hbm.at[pl.ds(0, CHUNK)],
                              sem_o[p]).wait()

    issue_ids(0, 0)
    issue_ids(1, 1)
    wait_ids(0)
    issue_gathers(0)

    def outer(ii, _):
        for p in range(2):
            ci = 2 * ii + p
            wait_gathers(p)

            @pl.when(ci + 2 < N_CHUNKS)
            def _():
                issue_ids(ci + 2, p)

            @pl.when(ci + 1 < N_CHUNKS)
            def _():
                wait_ids(1 - p)
                issue_gathers(1 - p)

            @pl.when(ci >= 2)
            def _():
                wait_out(p)

            def tok(i, c):
                for j in range(U):
                    t = i * U + j
                    x = [
                        x_v[p, t, pl.ds(k * 16, 16)]
                        + p_v[p, t, pl.ds(k * 16, 16)]
                        for k in range(K)
                    ]
                    s = x[0]
                    q = x[0] * x[0]
                    for k in range(1, K):
                        s = s + x[k]
                        q = q + x[k] * x[k]
                    tot = jnp.sum(s)
                    totq = jnp.sum(q)
                    mean = tot * (1.0 / D)
                    var = totq * (1.0 / D) - mean * mean
                    mean16 = jnp.full((16,), mean, jnp.float32)
                    rstd16 = _rsqrt(jnp.full((16,), var + EPS, jnp.float32))
                    for k in range(K):
                        y = (x[k] - mean16) * rstd16 * gk[k] + bk[k]
                        o_v[p, t, pl.ds(k * 16, 16)] = y
                return c

            lax.fori_loop(0, CHUNK // U, tok, 0)
            base = base0 + ci * CHUNK
            pltpu.async_copy(o_v.at[p], out_hbm.at[pl.ds(base, CHUNK)],
                             sem_o[p])
        return 0

    lax.fori_loop(0, N_CHUNKS // 2, outer, 0)
    wait_out(0)
    wait_out(1)


def kernel(masked_token_ids, visit_concept_orders, word_embeddings, pe,
           gamma, beta):
    ids = masked_token_ids.reshape(-1).astype(jnp.int32)
    orders = visit_concept_orders.reshape(-1).astype(jnp.int32)
    pe_s = (pe * (1.0 / math.sqrt(D))).astype(jnp.float32)
    out = _embed_ln(ids, orders, word_embeddings, pe_s,
                    gamma.astype(jnp.float32), beta.astype(jnp.float32))
    return out.reshape(masked_token_ids.shape[0], masked_token_ids.shape[1], D)
